# baseline (speedup 1.0000x reference)
import functools
import math

import jax
import jax.numpy as jnp
import numpy as np
from jax import lax
from jax.experimental import pallas as pl
from jax.experimental.pallas import tpu as pltpu
from jax.experimental.pallas import tpu_sc as plsc

VOCAB = 100000
D = 128
L_TOK = 24
B = 4096
MAX_LEN = 25


def _pe_const(max_len, d_model):
    position = np.arange(0, max_len, dtype=np.float32)[:, None]
    div_term = np.exp(
        np.arange(0, d_model, 2).astype(np.float32) * (-math.log(10000.0) / d_model)
    )
    pe = np.zeros((max_len, d_model), dtype=np.float32)
    pe[:, 0::2] = np.sin(position * div_term)
    pe[:, 1::2] = np.cos(position * div_term)
    return pe


_PE = _pe_const(MAX_LEN, D)

_INFO = plsc.get_sparse_core_info()
_NC = _INFO.num_cores
_NS = _INFO.num_subcores
_NW = _NC * _NS

_B_PER_W = B // _NW
_NB = 4
_CHUNKS = _B_PER_W // _NB
_TOK = _NB * L_TOK
_OROWS = _NB * MAX_LEN


def _sc_body(words_hbm, cls_hbm, noun_hbm, addt_hbm, spec_hbm,
             out_hbm,
             widx_v, sidx_v, spec_v,
             ob0, ob1, ob2, av0, av1, av2,
             sem_n0, sem_n1, sem_n2,
             sem_a0, sem_a1, sem_a2,
             sem_w0, sem_w1, sem_w2):
    wid = lax.axis_index("s") * _NC + lax.axis_index("c")
    b0w = wid * _B_PER_W
    obs = (ob0, ob1, ob2)
    avs = (av0, av1, av2)
    sems_n = (sem_n0, sem_n1, sem_n2)
    sems_a = (sem_a0, sem_a1, sem_a2)
    sems_w = (sem_w0, sem_w1, sem_w2)

    def fire_gathers(k, r):
        for j in range(_NB):
            pltpu.async_copy(
                noun_hbm.at[widx_v.at[k * _NB + j]],
                obs[r].at[pl.ds(j * MAX_LEN + 1, L_TOK)], sems_n[r])
        for j in range(_NB):
            pltpu.async_copy(
                addt_hbm.at[sidx_v.at[k * _NB + j]],
                avs[r].at[pl.ds(j * L_TOK, L_TOK)], sems_a[r])

    def drain_gathers(r):
        for j in range(_NB):
            pltpu.make_async_copy(
                noun_hbm.at[pl.ds(0, L_TOK)],
                obs[r].at[pl.ds(j * MAX_LEN + 1, L_TOK)], sems_n[r]).wait()
        for j in range(_NB):
            pltpu.make_async_copy(
                noun_hbm.at[pl.ds(0, L_TOK)],
                avs[r].at[pl.ds(j * L_TOK, L_TOK)], sems_a[r]).wait()

    def fire_writes(k, r):
        b0 = b0w + k * _NB
        for j in range(_NB):
            pltpu.async_copy(obs[r].at[pl.ds(j * MAX_LEN, MAX_LEN)],
                             out_hbm.at[b0 + j], sems_w[r])

    def drain_writes(r):
        for j in range(_NB):
            pltpu.make_async_copy(obs[r].at[pl.ds(j * MAX_LEN, MAX_LEN)],
                                  out_hbm.at[0], sems_w[r]).wait()

    def assemble(r):
        o, av = obs[r], avs[r]

        @plsc.parallel_loop(0, _NB)
        def batch(j):
            for l in range(L_TOK):
                orow = j * MAX_LEN + 1 + l
                tr = j * L_TOK + l
                for q in range(D // 16):
                    sl = pl.ds(q * 16, 16)
                    plsc.addupdate(o.at[orow, sl], av[tr, sl])

    def process(k, r, has_next2, drain_w):
        drain_gathers(r)
        assemble(r)
        pl.when(drain_w)(lambda: drain_writes((r + 2) % 3))
        if has_next2 is not None:
            pl.when(has_next2)(lambda: fire_gathers(k + 2, (r + 2) % 3))
        fire_writes(k, r)

    pltpu.sync_copy(words_hbm.at[pl.ds(b0w, _B_PER_W)], widx_v)
    pltpu.sync_copy(cls_hbm.at[pl.ds(b0w, _B_PER_W)], sidx_v)
    pltpu.sync_copy(spec_hbm, spec_v)
    iota16 = lax.broadcasted_iota(jnp.int32, (16,), 0)
    two_la = 2 * iota16
    two_lb = 2 * iota16 + 16

    def sidx_row(rr, carry):
        sla, slb = pl.ds(0, 16), pl.ds(8, 16)
        va = sidx_v[rr, sla]
        vb = sidx_v[rr, slb]
        sidx_v[rr, sla] = va + two_la
        sidx_v[rr, slb] = vb + two_lb
        return carry

    lax.fori_loop(0, _B_PER_W, sidx_row, 0)
    for ov in obs:
        for j in range(_NB):
            for q in range(D // 16):
                sl = pl.ds(q * 16, 16)
                ov[j * MAX_LEN, sl] = spec_v[0, sl]
    fire_gathers(0, 0)
    fire_gathers(1, 1)

    true_ = jnp.bool_(True)
    n_loop = _CHUNKS // 3 - (1 if _CHUNKS % 3 == 0 else 0)

    def triple(i, carry):
        k = 3 * i
        process(k + 0, 0, true_, k >= 1)
        process(k + 1, 1, true_, true_)
        process(k + 2, 2, 3 * i + 4 < _CHUNKS, true_)
        return carry

    lax.fori_loop(0, n_loop, triple, 0)
    for k in range(3 * n_loop, _CHUNKS):
        process(k, k % 3,
                true_ if k + 2 < _CHUNKS else None,
                true_ if k >= 1 else jnp.bool_(False))
    drain_writes((_CHUNKS - 1) % 3)


def kernel(words, classes, noun_table, class_table, special_emb):
    words_i = words.astype(jnp.int32)
    cls_i = classes.astype(jnp.int32)
    pe = jnp.asarray(_PE[:L_TOK])
    addt = (pe[:, None, :] + class_table[None, :, :]).reshape(2 * L_TOK, D)

    mesh = plsc.VectorSubcoreMesh(core_axis_name="c", subcore_axis_name="s")
    run = functools.partial(
        pl.kernel,
        mesh=mesh,
        compiler_params=pltpu.CompilerParams(needs_layout_passes=False),
        out_type=jax.ShapeDtypeStruct((B, MAX_LEN, D), jnp.float32),
        scratch_types=[
            pltpu.VMEM((_B_PER_W, L_TOK), jnp.int32),
            pltpu.VMEM((_B_PER_W, L_TOK), jnp.int32),
            pltpu.VMEM((1, D), jnp.float32),
            pltpu.VMEM((_OROWS, D), jnp.float32),
            pltpu.VMEM((_OROWS, D), jnp.float32),
            pltpu.VMEM((_OROWS, D), jnp.float32),
            pltpu.VMEM((_TOK, D), jnp.float32),
            pltpu.VMEM((_TOK, D), jnp.float32),
            pltpu.VMEM((_TOK, D), jnp.float32),
        ] + [pltpu.SemaphoreType.DMA] * 9,
    )(_sc_body)
    return run(words_i, cls_i, noun_table, addt, special_emb)


# speedup vs baseline: 1.6258x; 1.6258x over previous
import functools
import math

import jax
import jax.numpy as jnp
import numpy as np
from jax import lax
from jax.experimental import pallas as pl
from jax.experimental.pallas import tpu as pltpu
from jax.experimental.pallas import tpu_sc as plsc

VOCAB = 100000
D = 128
L_TOK = 24
B = 4096
MAX_LEN = 25


def _pe_const(max_len, d_model):
    position = np.arange(0, max_len, dtype=np.float32)[:, None]
    div_term = np.exp(
        np.arange(0, d_model, 2).astype(np.float32) * (-math.log(10000.0) / d_model)
    )
    pe = np.zeros((max_len, d_model), dtype=np.float32)
    pe[:, 0::2] = np.sin(position * div_term)
    pe[:, 1::2] = np.cos(position * div_term)
    return pe


_PE = _pe_const(MAX_LEN, D)

_INFO = plsc.get_sparse_core_info()
_NC = _INFO.num_cores
_NS = _INFO.num_subcores
_NW = _NC * _NS

_B_PER_W = B // _NW
_NB = 4
_CHUNKS = _B_PER_W // _NB
_TOK = _NB * L_TOK
_OROWS = _NB * MAX_LEN


def _sc_body(words_hbm, cls_hbm, noun_hbm, addt_hbm, spec_hbm,
             out_hbm,
             widx_v, sidx_v, spec_v, addt_sh,
             ob0, ob1, ob2, av0, av1, av2,
             sem_n0, sem_n1, sem_n2,
             sem_a0, sem_a1, sem_a2,
             sem_w0, sem_w1, sem_w2):
    wid = lax.axis_index("s") * _NC + lax.axis_index("c")
    b0w = wid * _B_PER_W
    obs = (ob0, ob1, ob2)
    avs = (av0, av1, av2)
    sems_n = (sem_n0, sem_n1, sem_n2)
    sems_a = (sem_a0, sem_a1, sem_a2)
    sems_w = (sem_w0, sem_w1, sem_w2)

    def fire_gathers(k, r):
        for j in range(_NB):
            pltpu.async_copy(
                noun_hbm.at[widx_v.at[k * _NB + j]],
                obs[r].at[pl.ds(j * MAX_LEN + 1, L_TOK)], sems_n[r])
        for j in range(_NB):
            pltpu.async_copy(
                addt_sh.at[sidx_v.at[k * _NB + j]],
                avs[r].at[pl.ds(j * L_TOK, L_TOK)], sems_a[r])

    def drain_gathers(r):
        for j in range(_NB):
            pltpu.make_async_copy(
                noun_hbm.at[pl.ds(0, L_TOK)],
                obs[r].at[pl.ds(j * MAX_LEN + 1, L_TOK)], sems_n[r]).wait()
        for j in range(_NB):
            pltpu.make_async_copy(
                noun_hbm.at[pl.ds(0, L_TOK)],
                avs[r].at[pl.ds(j * L_TOK, L_TOK)], sems_a[r]).wait()

    def fire_writes(k, r):
        b0 = b0w + k * _NB
        for j in range(_NB):
            pltpu.async_copy(obs[r].at[pl.ds(j * MAX_LEN, MAX_LEN)],
                             out_hbm.at[b0 + j], sems_w[r])

    def drain_writes(r):
        for j in range(_NB):
            pltpu.make_async_copy(obs[r].at[pl.ds(j * MAX_LEN, MAX_LEN)],
                                  out_hbm.at[0], sems_w[r]).wait()

    def assemble(r):
        o, av = obs[r], avs[r]

        @plsc.parallel_loop(0, _NB)
        def batch(j):
            for l in range(L_TOK):
                orow = j * MAX_LEN + 1 + l
                tr = j * L_TOK + l
                for q in range(D // 16):
                    sl = pl.ds(q * 16, 16)
                    plsc.addupdate(o.at[orow, sl], av[tr, sl])

    def process(k, r, has_next2, drain_w):
        drain_gathers(r)
        assemble(r)
        pl.when(drain_w)(lambda: drain_writes((r + 2) % 3))
        if has_next2 is not None:
            pl.when(has_next2)(lambda: fire_gathers(k + 2, (r + 2) % 3))
        fire_writes(k, r)

    pltpu.sync_copy(words_hbm.at[pl.ds(b0w, _B_PER_W)], widx_v)
    pltpu.sync_copy(cls_hbm.at[pl.ds(b0w, _B_PER_W)], sidx_v)
    pltpu.sync_copy(spec_hbm, spec_v)
    pl.when(lax.axis_index("s") == 0)(
        lambda: pltpu.sync_copy(addt_hbm, addt_sh))
    plsc.subcore_barrier()
    iota16 = lax.broadcasted_iota(jnp.int32, (16,), 0)
    two_la = 2 * iota16
    two_lb = 2 * iota16 + 16

    def sidx_row(rr, carry):
        sla, slb = pl.ds(0, 16), pl.ds(8, 16)
        va = sidx_v[rr, sla]
        vb = sidx_v[rr, slb]
        sidx_v[rr, sla] = va + two_la
        sidx_v[rr, slb] = vb + two_lb
        return carry

    lax.fori_loop(0, _B_PER_W, sidx_row, 0)
    for ov in obs:
        for j in range(_NB):
            for q in range(D // 16):
                sl = pl.ds(q * 16, 16)
                ov[j * MAX_LEN, sl] = spec_v[0, sl]
    fire_gathers(0, 0)
    fire_gathers(1, 1)

    true_ = jnp.bool_(True)
    n_loop = _CHUNKS // 3 - (1 if _CHUNKS % 3 == 0 else 0)

    def triple(i, carry):
        k = 3 * i
        process(k + 0, 0, true_, k >= 1)
        process(k + 1, 1, true_, true_)
        process(k + 2, 2, 3 * i + 4 < _CHUNKS, true_)
        return carry

    lax.fori_loop(0, n_loop, triple, 0)
    for k in range(3 * n_loop, _CHUNKS):
        process(k, k % 3,
                true_ if k + 2 < _CHUNKS else None,
                true_ if k >= 1 else jnp.bool_(False))
    drain_writes((_CHUNKS - 1) % 3)


def kernel(words, classes, noun_table, class_table, special_emb):
    words_i = words.astype(jnp.int32)
    cls_i = classes.astype(jnp.int32)
    pe = jnp.asarray(_PE[:L_TOK])
    addt = (pe[:, None, :] + class_table[None, :, :]).reshape(2 * L_TOK, D)

    mesh = plsc.VectorSubcoreMesh(core_axis_name="c", subcore_axis_name="s")
    run = functools.partial(
        pl.kernel,
        mesh=mesh,
        compiler_params=pltpu.CompilerParams(needs_layout_passes=False),
        out_type=jax.ShapeDtypeStruct((B, MAX_LEN, D), jnp.float32),
        scratch_types=[
            pltpu.VMEM((_B_PER_W, L_TOK), jnp.int32),
            pltpu.VMEM((_B_PER_W, L_TOK), jnp.int32),
            pltpu.VMEM((1, D), jnp.float32),
            pltpu.VMEM_SHARED((2 * L_TOK, D), jnp.float32),
            pltpu.VMEM((_OROWS, D), jnp.float32),
            pltpu.VMEM((_OROWS, D), jnp.float32),
            pltpu.VMEM((_OROWS, D), jnp.float32),
            pltpu.VMEM((_TOK, D), jnp.float32),
            pltpu.VMEM((_TOK, D), jnp.float32),
            pltpu.VMEM((_TOK, D), jnp.float32),
        ] + [pltpu.SemaphoreType.DMA] * 9,
    )(_sc_body)
    return run(words_i, cls_i, noun_table, addt, special_emb)
